# baseline (speedup 1.0000x reference)
import functools

import jax
import jax.numpy as jnp
from jax import lax
from jax.experimental import pallas as pl
from jax.experimental.pallas import tpu as pltpu
from jax.experimental.pallas import tpu_sc as plsc

_DP = 128
_B_TOTAL = 4096 * 200
_NW = 32
_B_PER_W = _B_TOTAL // _NW
_CHUNK = 320
_N_CHUNKS = _B_PER_W // _CHUNK
_N_PAIRS = _N_CHUNKS // 2

_mesh = plsc.VectorSubcoreMesh(core_axis_name="c", subcore_axis_name="s")

_TBLK = 1024
_N_ROWS = 1000000


def _transpose_body(x_ref, o_ref):
    xt = jax.lax.dot_general(
        x_ref[...], jnp.eye(64, dtype=jnp.float32),
        dimension_numbers=(((0,), (0,)), ((), ())),
        preferred_element_type=jnp.float32,
        precision=jax.lax.Precision.HIGHEST,
    )
    o_ref[:, 0:64] = xt


_transpose_tc = pl.pallas_call(
    _transpose_body,
    grid=((_N_ROWS + _TBLK - 1) // _TBLK,),
    in_specs=[pl.BlockSpec((64, _TBLK), lambda j: (0, j))],
    out_specs=pl.BlockSpec((_TBLK, _DP), lambda j: (j, 0)),
    out_shape=jax.ShapeDtypeStruct((_N_ROWS, _DP), jnp.float32),
)


@functools.partial(
    pl.kernel,
    out_type=jax.ShapeDtypeStruct((_B_TOTAL, _DP), jnp.float32),
    mesh=_mesh,
    scratch_types=[
        pltpu.VMEM((_B_PER_W,), jnp.int32),
        pltpu.VMEM((_CHUNK, _DP), jnp.float32),
        pltpu.VMEM((_CHUNK, _DP), jnp.float32),
        pltpu.SemaphoreType.DMA,
        pltpu.SemaphoreType.DMA,
        pltpu.SemaphoreType.DMA,
        pltpu.SemaphoreType.DMA,
    ],
    compiler_params=pltpu.CompilerParams(use_tc_tiling_on_sc=False),
)
def _gather_kernel(table_hbm, idx_hbm, out_hbm, idx_v, rows0, rows1,
                   g0, g1, s0, s1):
    wid = lax.axis_index("s") * 2 + lax.axis_index("c")
    base = wid * _B_PER_W
    pltpu.sync_copy(idx_hbm.at[pl.ds(base, _B_PER_W)], idx_v)

    def g_start(i, buf, sem):
        pltpu.async_copy(table_hbm.at[idx_v.at[pl.ds(i * _CHUNK, _CHUNK)]],
                         buf, sem)

    def g_wait(buf, sem):
        pltpu.make_async_copy(table_hbm.at[idx_v.at[pl.ds(0, _CHUNK)]],
                              buf, sem).wait()

    def s_start(i, buf, sem):
        pltpu.async_copy(buf.at[:, pl.ds(0, 64)],
                         out_hbm.at[pl.ds(base + i * _CHUNK, _CHUNK),
                                    pl.ds(0, 64)],
                         sem)

    def s_wait(buf, sem):
        pltpu.make_async_copy(buf.at[:, pl.ds(0, 64)],
                              out_hbm.at[pl.ds(base, _CHUNK), pl.ds(0, 64)],
                              sem).wait()

    g_start(0, rows0, g0)
    g_start(1, rows1, g1)

    def body(p, carry):
        i0 = 2 * p
        g_wait(rows0, g0)
        s_start(i0, rows0, s0)
        g_wait(rows1, g1)
        s_start(i0 + 1, rows1, s1)

        @pl.when(p + 1 < _N_PAIRS)
        def _prefetch():
            s_wait(rows0, s0)
            g_start(i0 + 2, rows0, g0)
            s_wait(rows1, s1)
            g_start(i0 + 3, rows1, g1)

        return carry

    lax.fori_loop(0, _N_PAIRS, body, 0)
    s_wait(rows0, s0)
    s_wait(rows1, s1)


def kernel(weight, token_ids):
    wpad = _transpose_tc(weight.T)
    flat_ids = token_ids.reshape(-1).astype(jnp.int32)
    outp = _gather_kernel(wpad, flat_ids)
    out3 = outp.reshape(token_ids.shape + (_DP,))
    return out3[:, :, : weight.shape[1]]


# speedup vs baseline: 1.4600x; 1.4600x over previous
import functools

import jax
import jax.numpy as jnp
from jax import lax
from jax.experimental import pallas as pl
from jax.experimental.pallas import tpu as pltpu
from jax.experimental.pallas import tpu_sc as plsc

_DP = 128
_B_TOTAL = 4096 * 200
_NW = 32
_B_PER_W = _B_TOTAL // _NW
_CHUNK = 320
_N_CHUNKS = _B_PER_W // _CHUNK
_N_PAIRS = _N_CHUNKS // 2

_mesh = plsc.VectorSubcoreMesh(core_axis_name="c", subcore_axis_name="s")

_N_ROWS = 1000000
_NBLK = 7812
_TAIL = _NBLK * 128
_BPT = 245
_L_ROWS = _TAIL + 128


@functools.partial(
    pl.kernel,
    out_type=jax.ShapeDtypeStruct((_L_ROWS, _DP), jnp.float32),
    mesh=_mesh,
    scratch_types=[
        pltpu.VMEM((64, 128), jnp.float32),
        pltpu.VMEM((64, 128), jnp.float32),
        pltpu.VMEM((128, _DP), jnp.float32),
        pltpu.VMEM((128, _DP), jnp.float32),
        pltpu.VMEM((64, 64), jnp.float32),
        pltpu.SemaphoreType.DMA,
        pltpu.SemaphoreType.DMA,
        pltpu.SemaphoreType.DMA,
        pltpu.SemaphoreType.DMA,
    ],
)
def _transpose_sc(wt_hbm, wtail_hbm, l_hbm, bin0, bin1, bout0, bout1,
                  bin_t, gi0, gi1, so0, so1):
    wid = lax.axis_index("s") * 2 + lax.axis_index("c")

    def blk(k):
        return wid + 32 * k

    def in_start(k, bin_b, sem):
        c = blk(k)

        @pl.when(c < _NBLK)
        def _():
            pltpu.async_copy(wt_hbm.at[:, pl.ds(c * 128, 128)], bin_b, sem)

    def in_wait(k, bin_b, sem):
        c = blk(k)

        @pl.when(c < _NBLK)
        def _():
            pltpu.make_async_copy(wt_hbm.at[:, pl.ds(0, 128)], bin_b,
                                  sem).wait()

    def out_start(k, bout_b, sem):
        c = blk(k)

        @pl.when(c < _NBLK)
        def _():
            pltpu.async_copy(bout_b, l_hbm.at[pl.ds(c * 128, 128)], sem)

    def out_wait(k, bout_b, sem):
        c = blk(k)

        @pl.when(c < _NBLK)
        def _():
            pltpu.make_async_copy(bout_b, l_hbm.at[pl.ds(0, 128)],
                                  sem).wait()

    iotav = lax.iota(jnp.int32, 16)
    rows = [iotav + 16 * k for k in range(4)]

    def transpose(bin_b, bout_b):
        def body(i, carry):
            coli = jnp.full((16,), i, jnp.int32)
            for k in range(4):
                v = plsc.load_gather(bin_b, [rows[k], coli])
                plsc.store_scatter(bout_b, [coli, rows[k]], v)
            return carry

        lax.fori_loop(0, 128, body, 0)

    def step(k, bin_b, bout_b, gi, so):
        c = blk(k)

        @pl.when(c < _NBLK)
        def _():
            in_wait(k, bin_b, gi)

            @pl.when(k >= 2)
            def _w():
                out_wait(k - 2, bout_b, so)

            transpose(bin_b, bout_b)
            out_start(k, bout_b, so)
            in_start(k + 2, bin_b, gi)

    @pl.when(wid == 31)
    def _tail():
        pltpu.sync_copy(wtail_hbm, bin_t)

        def tbody(i, carry):
            coli = jnp.full((16,), i, jnp.int32)
            for k in range(4):
                v = plsc.load_gather(bin_t, [rows[k], coli])
                plsc.store_scatter(bout0, [coli, rows[k]], v)
            return carry

        lax.fori_loop(0, 64, tbody, 0)
        pltpu.sync_copy(bout0, l_hbm.at[pl.ds(_TAIL, 128)])

    in_start(0, bin0, gi0)
    in_start(1, bin1, gi1)

    def body(p, carry):
        step(2 * p, bin0, bout0, gi0, so0)
        step(2 * p + 1, bin1, bout1, gi1, so1)
        return carry

    lax.fori_loop(0, (_BPT + 1) // 2, body, 0)
    out_wait(_BPT - 1, bout0, so0)
    out_wait(_BPT - 2, bout1, so1)


@functools.partial(
    pl.kernel,
    out_type=jax.ShapeDtypeStruct((_B_TOTAL, _DP), jnp.float32),
    mesh=_mesh,
    scratch_types=[
        pltpu.VMEM((_B_PER_W,), jnp.int32),
        pltpu.VMEM((_CHUNK, 64), jnp.float32),
        pltpu.VMEM((_CHUNK, 64), jnp.float32),
        pltpu.SemaphoreType.DMA,
        pltpu.SemaphoreType.DMA,
        pltpu.SemaphoreType.DMA,
        pltpu.SemaphoreType.DMA,
    ],
    compiler_params=pltpu.CompilerParams(use_tc_tiling_on_sc=False),
)
def _gather_kernel(table_hbm, idx_hbm, out_hbm, idx_v, rows0, rows1,
                   g0, g1, s0, s1):
    wid = lax.axis_index("s") * 2 + lax.axis_index("c")
    base = wid * _B_PER_W
    pltpu.sync_copy(idx_hbm.at[pl.ds(base, _B_PER_W)], idx_v)

    def g_start(i, buf, sem):
        pltpu.async_copy(table_hbm.at[idx_v.at[pl.ds(i * _CHUNK, _CHUNK)]],
                         buf, sem)

    def g_wait(buf, sem):
        pltpu.make_async_copy(table_hbm.at[idx_v.at[pl.ds(0, _CHUNK)]],
                              buf, sem).wait()

    def s_start(i, buf, sem):
        pltpu.async_copy(buf,
                         out_hbm.at[pl.ds(base + i * _CHUNK, _CHUNK),
                                    pl.ds(0, 64)],
                         sem)

    def s_wait(buf, sem):
        pltpu.make_async_copy(buf,
                              out_hbm.at[pl.ds(base, _CHUNK), pl.ds(0, 64)],
                              sem).wait()

    g_start(0, rows0, g0)
    g_start(1, rows1, g1)

    def body(p, carry):
        i0 = 2 * p
        g_wait(rows0, g0)
        s_start(i0, rows0, s0)
        g_wait(rows1, g1)
        s_start(i0 + 1, rows1, s1)

        @pl.when(p + 1 < _N_PAIRS)
        def _prefetch():
            s_wait(rows0, s0)
            g_start(i0 + 2, rows0, g0)
            s_wait(rows1, s1)
            g_start(i0 + 3, rows1, g1)

        return carry

    lax.fori_loop(0, _N_PAIRS, body, 0)
    s_wait(rows0, s0)
    s_wait(rows1, s1)


def kernel(weight, token_ids):
    wlin = jax.lax.optimization_barrier(weight.reshape(-1))
    wtab = wlin.reshape(_N_ROWS, 64)
    flat_ids = token_ids.reshape(-1).astype(jnp.int32)
    outp = _gather_kernel(wtab, flat_ids)
    out3 = outp.reshape(token_ids.shape + (_DP,))
    return out3[:, :, : weight.shape[1]]


# speedup vs baseline: 1.4691x; 1.0063x over previous
import functools

import jax
import jax.numpy as jnp
from jax import lax
from jax.experimental import pallas as pl
from jax.experimental.pallas import tpu as pltpu
from jax.experimental.pallas import tpu_sc as plsc

_DP = 128
_B_TOTAL = 4096 * 200
_NW = 32
_B_PER_W = _B_TOTAL // _NW
_CHUNK = 640
_N_CHUNKS = _B_PER_W // _CHUNK
_N_PAIRS = _N_CHUNKS // 2

_mesh = plsc.VectorSubcoreMesh(core_axis_name="c", subcore_axis_name="s")


@functools.partial(
    pl.kernel,
    out_type=jax.ShapeDtypeStruct((_B_TOTAL, _DP), jnp.float32),
    mesh=_mesh,
    scratch_types=[
        pltpu.VMEM((_B_PER_W,), jnp.int32),
        pltpu.VMEM((_CHUNK, 64), jnp.float32),
        pltpu.VMEM((_CHUNK, 64), jnp.float32),
        pltpu.SemaphoreType.DMA,
        pltpu.SemaphoreType.DMA,
        pltpu.SemaphoreType.DMA,
        pltpu.SemaphoreType.DMA,
    ],
    compiler_params=pltpu.CompilerParams(use_tc_tiling_on_sc=False),
)
def _gather_kernel(table_hbm, idx_hbm, out_hbm, idx_v, rows0, rows1,
                   g0, g1, s0, s1):
    wid = lax.axis_index("s") * 2 + lax.axis_index("c")
    base = wid * _B_PER_W
    pltpu.sync_copy(idx_hbm.at[pl.ds(base, _B_PER_W)], idx_v)

    def g_start(i, buf, sem):
        pltpu.async_copy(table_hbm.at[idx_v.at[pl.ds(i * _CHUNK, _CHUNK)]],
                         buf, sem)

    def g_wait(buf, sem):
        pltpu.make_async_copy(table_hbm.at[idx_v.at[pl.ds(0, _CHUNK)]],
                              buf, sem).wait()

    def s_start(i, buf, sem):
        pltpu.async_copy(buf,
                         out_hbm.at[pl.ds(base + i * _CHUNK, _CHUNK),
                                    pl.ds(0, 64)],
                         sem)

    def s_wait(buf, sem):
        pltpu.make_async_copy(buf,
                              out_hbm.at[pl.ds(base, _CHUNK), pl.ds(0, 64)],
                              sem).wait()

    g_start(0, rows0, g0)
    g_start(1, rows1, g1)

    def body(p, carry):
        i0 = 2 * p
        g_wait(rows0, g0)
        s_start(i0, rows0, s0)
        g_wait(rows1, g1)
        s_start(i0 + 1, rows1, s1)

        @pl.when(p + 1 < _N_PAIRS)
        def _prefetch():
            s_wait(rows0, s0)
            g_start(i0 + 2, rows0, g0)
            s_wait(rows1, s1)
            g_start(i0 + 3, rows1, g1)

        return carry

    lax.fori_loop(0, _N_PAIRS, body, 0)
    s_wait(rows0, s0)
    s_wait(rows1, s1)


def kernel(weight, token_ids):
    wlin = jax.lax.optimization_barrier(weight.reshape(-1))
    wtab = wlin.reshape(1000000, 64)
    flat_ids = token_ids.reshape(-1).astype(jnp.int32)
    outp = _gather_kernel(wtab, flat_ids)
    out3 = outp.reshape(token_ids.shape + (_DP,))
    return out3[:, :, : weight.shape[1]]
